# baseline (speedup 1.0000x reference)
import functools

import jax
import jax.numpy as jnp
from jax import lax
from jax.experimental import pallas as pl
from jax.experimental.pallas import tpu as pltpu
from jax.experimental.pallas import tpu_sc as plsc


def _sc_gather_body(rows_per, lut_hbm, ids_hbm, out_hbm, idx_v, val_v, sem):
    nc = plsc.get_sparse_core_info().num_cores
    wid = (lax.axis_index("s") * jnp.int32(nc)
           + lax.axis_index("c")).astype(jnp.int32)
    base = wid * jnp.int32(rows_per)
    pltpu.sync_copy(ids_hbm.at[pl.ds(base, rows_per)], idx_v)
    copies = [
        pltpu.async_copy(
            lut_hbm.at[idx_v.at[jnp.int32(j)]],
            val_v.at[jnp.int32(j)], sem)
        for j in range(rows_per)
    ]
    for c in copies:
        c.wait()
    pltpu.sync_copy(val_v, out_hbm.at[pl.ds(base, rows_per)])


def _sc_gather(lut_words, ids_2d):
    rows, width = ids_2d.shape
    info = plsc.get_sparse_core_info()
    rows_per = rows // (info.num_cores * info.num_subcores)
    mesh = plsc.VectorSubcoreMesh(core_axis_name="c", subcore_axis_name="s")

    return pl.kernel(
        functools.partial(_sc_gather_body, rows_per),
        out_type=jax.ShapeDtypeStruct((rows, width), jnp.int32),
        mesh=mesh,
        scratch_types=[
            pltpu.VMEM((rows_per, width), jnp.int32),
            pltpu.VMEM((rows_per, width), jnp.int32),
            pltpu.SemaphoreType.DMA,
        ],
    )(lut_words, ids_2d)


def _hash_body(max_ngram, n_head, b, t, aux_ref, s_ref, out_ref):
    mask16 = jnp.uint32(0xFFFF)
    view = s_ref[...].reshape(b, t)
    pad = aux_ref[3 * max_ngram + n_head * (max_ngram - 1)]

    prods = []
    for k in range(max_ngram):
        if k == 0:
            s = view.astype(jnp.uint32)
        else:
            s = jnp.concatenate(
                [jnp.full((b, k), pad, jnp.int32), view[:, :t - k]],
                axis=1).astype(jnp.uint32)
        s0 = s & mask16
        s1_nz = (s >> 16) > 0
        mk0 = aux_ref[3 * k].astype(jnp.uint32)
        mk1 = aux_ref[3 * k + 1].astype(jnp.uint32)
        mk2 = aux_ref[3 * k + 2].astype(jnp.uint32)
        a0 = s0 * mk0
        a1 = s0 * mk1
        a2 = s0 * mk2
        b0 = jnp.where(s1_nz, mk0, jnp.uint32(0))
        b1 = jnp.where(s1_nz, mk1, jnp.uint32(0))
        b2 = jnp.where(s1_nz, mk2, jnp.uint32(0))
        c0 = a0 & mask16
        t1 = (a0 >> 16) + (a1 & mask16) + b0
        t2 = (t1 >> 16) + (a1 >> 16) + (a2 & mask16) + b1
        t3 = (t2 >> 16) + (a2 >> 16) + b2
        lo = c0 | ((t1 & mask16) << 16)
        hi = (t2 & mask16) | ((t3 & mask16) << 16)
        prods.append((hi, lo))

    mix_hi, mix_lo = prods[0]
    idx = 0
    for n in range(2, max_ngram + 1):
        mix_hi = mix_hi ^ prods[n - 1][0]
        mix_lo = mix_lo ^ prods[n - 1][1]
        hi_s = mix_hi.astype(jnp.int32)
        l1 = (mix_lo >> 16).astype(jnp.int32)
        l0 = (mix_lo & mask16).astype(jnp.int32)
        for _ in range(n_head):
            p = aux_ref[3 * max_ngram + idx]
            inv = jnp.float32(1.0) / p.astype(jnp.float32)

            def fmod31(y):
                q = (y.astype(jnp.float32) * inv).astype(jnp.int32)
                r = y - q * p
                r = jnp.where(r < 0, r + p, r)
                return jnp.where(r >= p, r - p, r)

            acc = fmod31(hi_s)
            acc = fmod31(acc << 14)
            acc = fmod31(((acc << 2) + l1) << 12)
            acc = fmod31((acc << 4) + l0)
            out_ref[idx] = acc.reshape(out_ref.shape[1], out_ref.shape[2])
            idx += 1


def _tc_hash(max_ngram, n_head, b, t, aux32, s_2d, interpret=False):
    n_out = (max_ngram - 1) * n_head
    return pl.pallas_call(
        functools.partial(_hash_body, max_ngram, n_head, b, t),
        out_shape=jax.ShapeDtypeStruct(
            (n_out, s_2d.shape[0], s_2d.shape[1]), jnp.int32),
        in_specs=[
            pl.BlockSpec(memory_space=pltpu.SMEM),
            pl.BlockSpec(memory_space=pltpu.VMEM),
        ],
        out_specs=pl.BlockSpec(memory_space=pltpu.VMEM),
        interpret=interpret,
    )(aux32, s_2d)


def _build_aux(multipliers, prime_mods, pad_id, max_ngram):
    limbs = jnp.stack(
        [(multipliers[k] >> (16 * j)) & 0xFFFF
         for k in range(max_ngram) for j in range(3)])
    pad64 = jnp.asarray(pad_id).astype(jnp.int64)
    return jnp.concatenate([limbs, prime_mods, pad64[None]]).astype(jnp.int32)


def kernel(input_ids, lookup_table, multipliers, prime_mods, pad_id):
    b, t = input_ids.shape
    max_ngram = multipliers.shape[0]
    n_head = prime_mods.shape[0] // (max_ngram - 1)

    vmax = lookup_table.shape[0] - 1
    idsw = (jnp.clip(input_ids, 0, vmax) * 2).astype(jnp.int32)
    lut_words = jax.lax.bitcast_convert_type(
        lookup_table, jnp.int32).reshape(-1)
    aux32 = _build_aux(multipliers, prime_mods, pad_id, max_ngram)

    s_2d = _sc_gather(lut_words, idsw.reshape(-1, 128))
    out = _tc_hash(max_ngram, n_head, b, t, aux32, s_2d)
    out = out.reshape(prime_mods.shape[0], b, t)
    return jnp.transpose(out, (1, 2, 0)).astype(jnp.int64)


# speedup vs baseline: 2.4477x; 2.4477x over previous
import functools

import jax
import jax.numpy as jnp
from jax import lax
from jax.experimental import pallas as pl
from jax.experimental.pallas import tpu as pltpu
from jax.experimental.pallas import tpu_sc as plsc


def _sc_gather_body(rows_per, lut_hbm, ids_hbm, out_hbm, idx_v, val_v, sem):
    nc = plsc.get_sparse_core_info().num_cores
    wid = (lax.axis_index("s") * jnp.int32(nc)
           + lax.axis_index("c")).astype(jnp.int32)
    base = wid * jnp.int32(rows_per)
    pltpu.sync_copy(ids_hbm.at[pl.ds(base, rows_per)], idx_v)
    copies = [
        pltpu.async_copy(
            lut_hbm.at[idx_v.at[jnp.int32(j)]],
            val_v.at[jnp.int32(j)], sem)
        for j in range(rows_per)
    ]
    for c in copies:
        c.wait()
    pltpu.sync_copy(val_v, out_hbm.at[pl.ds(base, rows_per)])


def _sc_gather(lut32, ids_2d):
    rows, width = ids_2d.shape
    info = plsc.get_sparse_core_info()
    rows_per = rows // (info.num_cores * info.num_subcores)
    mesh = plsc.VectorSubcoreMesh(core_axis_name="c", subcore_axis_name="s")

    return pl.kernel(
        functools.partial(_sc_gather_body, rows_per),
        out_type=jax.ShapeDtypeStruct((rows, width), jnp.int32),
        mesh=mesh,
        scratch_types=[
            pltpu.VMEM((rows_per, width), jnp.int32),
            pltpu.VMEM((rows_per, width), jnp.int32),
            pltpu.SemaphoreType.DMA,
        ],
    )(lut32, ids_2d)


def _hash_body(max_ngram, n_head, b, t, aux_ref, s_ref, out_ref):
    mask16 = jnp.uint32(0xFFFF)
    view = s_ref[...].reshape(b, t)
    pad = aux_ref[3 * max_ngram + n_head * (max_ngram - 1)]

    prods = []
    for k in range(max_ngram):
        if k == 0:
            s = view.astype(jnp.uint32)
        else:
            s = jnp.concatenate(
                [jnp.full((b, k), pad, jnp.int32), view[:, :t - k]],
                axis=1).astype(jnp.uint32)
        s0 = s & mask16
        s1_nz = (s >> 16) > 0
        mk0 = aux_ref[3 * k].astype(jnp.uint32)
        mk1 = aux_ref[3 * k + 1].astype(jnp.uint32)
        mk2 = aux_ref[3 * k + 2].astype(jnp.uint32)
        a0 = s0 * mk0
        a1 = s0 * mk1
        a2 = s0 * mk2
        b0 = jnp.where(s1_nz, mk0, jnp.uint32(0))
        b1 = jnp.where(s1_nz, mk1, jnp.uint32(0))
        b2 = jnp.where(s1_nz, mk2, jnp.uint32(0))
        c0 = a0 & mask16
        t1 = (a0 >> 16) + (a1 & mask16) + b0
        t2 = (t1 >> 16) + (a1 >> 16) + (a2 & mask16) + b1
        t3 = (t2 >> 16) + (a2 >> 16) + b2
        lo = c0 | ((t1 & mask16) << 16)
        hi = (t2 & mask16) | ((t3 & mask16) << 16)
        prods.append((hi, lo))

    mix_hi, mix_lo = prods[0]
    idx = 0
    for n in range(2, max_ngram + 1):
        mix_hi = mix_hi ^ prods[n - 1][0]
        mix_lo = mix_lo ^ prods[n - 1][1]
        hi_s = mix_hi.astype(jnp.int32)
        l1 = (mix_lo >> 16).astype(jnp.int32)
        l0 = (mix_lo & mask16).astype(jnp.int32)
        for _ in range(n_head):
            p = aux_ref[3 * max_ngram + idx]
            inv = jnp.float32(1.0) / p.astype(jnp.float32)

            def fmod31(y):
                q = (y.astype(jnp.float32) * inv).astype(jnp.int32)
                r = y - q * p
                r = jnp.where(r < 0, r + p, r)
                return jnp.where(r >= p, r - p, r)

            acc = fmod31(hi_s)
            acc = fmod31(acc << 14)
            acc = fmod31(((acc << 2) + l1) << 12)
            acc = fmod31((acc << 4) + l0)
            out_ref[idx] = acc.reshape(out_ref.shape[1], out_ref.shape[2])
            idx += 1


def _tc_hash(max_ngram, n_head, b, t, aux32, s_2d, interpret=False):
    n_out = (max_ngram - 1) * n_head
    return pl.pallas_call(
        functools.partial(_hash_body, max_ngram, n_head, b, t),
        out_shape=jax.ShapeDtypeStruct(
            (n_out, s_2d.shape[0], s_2d.shape[1]), jnp.int32),
        in_specs=[
            pl.BlockSpec(memory_space=pltpu.SMEM),
            pl.BlockSpec(memory_space=pltpu.VMEM),
        ],
        out_specs=pl.BlockSpec(memory_space=pltpu.VMEM),
        interpret=interpret,
    )(aux32, s_2d)


def _build_aux(multipliers, prime_mods, pad_id, max_ngram):
    limbs = jnp.stack(
        [(multipliers[k] >> (16 * j)) & 0xFFFF
         for k in range(max_ngram) for j in range(3)])
    pad64 = jnp.asarray(pad_id).astype(jnp.int64)
    return jnp.concatenate([limbs, prime_mods, pad64[None]]).astype(jnp.int32)


def kernel(input_ids, lookup_table, multipliers, prime_mods, pad_id):
    b, t = input_ids.shape
    max_ngram = multipliers.shape[0]
    n_head = prime_mods.shape[0] // (max_ngram - 1)

    vmax = lookup_table.shape[0] - 1
    ids32 = jnp.clip(input_ids, 0, vmax).astype(jnp.int32)
    lut32 = lookup_table.astype(jnp.int32)
    aux32 = _build_aux(multipliers, prime_mods, pad_id, max_ngram)

    s_2d = _sc_gather(lut32, ids32.reshape(-1, 128))
    out = _tc_hash(max_ngram, n_head, b, t, aux32, s_2d)
    out = out.reshape(prime_mods.shape[0], b, t)
    return jnp.transpose(out, (1, 2, 0)).astype(jnp.int64)


# speedup vs baseline: 2.4564x; 1.0036x over previous
import functools

import jax
import jax.numpy as jnp
from jax import lax
from jax.experimental import pallas as pl
from jax.experimental.pallas import tpu as pltpu
from jax.experimental.pallas import tpu_sc as plsc


def _sc_gather_body(rows_per, lut_hbm, ids_hbm, out_hbm, idx_v, val_v, sem):
    nc = plsc.get_sparse_core_info().num_cores
    wid = (lax.axis_index("s") * jnp.int32(nc)
           + lax.axis_index("c")).astype(jnp.int32)
    base = wid * jnp.int32(rows_per)
    pltpu.sync_copy(ids_hbm.at[pl.ds(base, rows_per)], idx_v)
    copies = [
        pltpu.async_copy(
            lut_hbm.at[idx_v.at[jnp.int32(j)]],
            val_v.at[jnp.int32(j)], sem)
        for j in range(rows_per)
    ]
    for c in copies:
        c.wait()
    pltpu.sync_copy(val_v, out_hbm.at[pl.ds(base, rows_per)])


def _sc_gather(lut32, ids_2d):
    rows, width = ids_2d.shape
    info = plsc.get_sparse_core_info()
    rows_per = rows // (info.num_cores * info.num_subcores)
    mesh = plsc.VectorSubcoreMesh(core_axis_name="c", subcore_axis_name="s")

    return pl.kernel(
        functools.partial(_sc_gather_body, rows_per),
        out_type=jax.ShapeDtypeStruct((rows, width), jnp.int32),
        mesh=mesh,
        scratch_types=[
            pltpu.VMEM((rows_per, width), jnp.int32),
            pltpu.VMEM((rows_per, width), jnp.int32),
            pltpu.SemaphoreType.DMA,
        ],
    )(lut32, ids_2d)


def _hash_body(max_ngram, n_head, b, t, aux_ref, s_ref, out_ref):
    mask16 = jnp.uint32(0xFFFF)
    view = s_ref[...].reshape(b, t)
    pad = aux_ref[3 * max_ngram + n_head * (max_ngram - 1)]

    prods = []
    for k in range(max_ngram):
        if k == 0:
            s = view.astype(jnp.uint32)
        else:
            s = jnp.concatenate(
                [jnp.full((b, k), pad, jnp.int32), view[:, :t - k]],
                axis=1).astype(jnp.uint32)
        s0 = s & mask16
        s1_nz = (s >> 16) > 0
        mk0 = aux_ref[3 * k].astype(jnp.uint32)
        mk1 = aux_ref[3 * k + 1].astype(jnp.uint32)
        mk2 = aux_ref[3 * k + 2].astype(jnp.uint32)
        a0 = s0 * mk0
        a1 = s0 * mk1
        a2 = s0 * mk2
        b0 = jnp.where(s1_nz, mk0, jnp.uint32(0))
        b1 = jnp.where(s1_nz, mk1, jnp.uint32(0))
        b2 = jnp.where(s1_nz, mk2, jnp.uint32(0))
        c0 = a0 & mask16
        t1 = (a0 >> 16) + (a1 & mask16) + b0
        t2 = (t1 >> 16) + (a1 >> 16) + (a2 & mask16) + b1
        t3 = (t2 >> 16) + (a2 >> 16) + b2
        lo = c0 | ((t1 & mask16) << 16)
        hi = (t2 & mask16) | ((t3 & mask16) << 16)
        prods.append((hi, lo))

    mix_hi, mix_lo = prods[0]
    idx = 0
    for n in range(2, max_ngram + 1):
        mix_hi = mix_hi ^ prods[n - 1][0]
        mix_lo = mix_lo ^ prods[n - 1][1]
        hi_s = mix_hi.astype(jnp.int32)
        l1 = (mix_lo >> 16).astype(jnp.int32)
        l0 = (mix_lo & mask16).astype(jnp.int32)
        for _ in range(n_head):
            p = aux_ref[3 * max_ngram + idx]
            inv = jnp.float32(1.0) / p.astype(jnp.float32)

            def fmod31(y):
                q = (y.astype(jnp.float32) * inv).astype(jnp.int32)
                r = y - q * p
                r = jnp.where(r < 0, r + p, r)
                return jnp.where(r >= p, r - p, r)

            acc = fmod31(hi_s)
            acc = fmod31(acc << 14)
            acc = fmod31(((acc << 2) + l1) << 12)
            acc = fmod31((acc << 4) + l0)
            out_ref[idx] = acc
            idx += 1


def _tc_hash(max_ngram, n_head, b, t, aux32, s_2d, interpret=False):
    n_out = (max_ngram - 1) * n_head
    return pl.pallas_call(
        functools.partial(_hash_body, max_ngram, n_head, b, t),
        out_shape=jax.ShapeDtypeStruct((n_out, b, t), jnp.int32),
        in_specs=[
            pl.BlockSpec(memory_space=pltpu.SMEM),
            pl.BlockSpec(memory_space=pltpu.VMEM),
        ],
        out_specs=pl.BlockSpec(memory_space=pltpu.VMEM),
        interpret=interpret,
    )(aux32, s_2d)


def _build_aux(multipliers, prime_mods, pad_id, max_ngram):
    limbs = jnp.stack(
        [(multipliers[k] >> (16 * j)) & 0xFFFF
         for k in range(max_ngram) for j in range(3)])
    pad64 = jnp.asarray(pad_id).astype(jnp.int64)
    return jnp.concatenate([limbs, prime_mods, pad64[None]]).astype(jnp.int32)


def kernel(input_ids, lookup_table, multipliers, prime_mods, pad_id):
    b, t = input_ids.shape
    max_ngram = multipliers.shape[0]
    n_head = prime_mods.shape[0] // (max_ngram - 1)

    vmax = lookup_table.shape[0] - 1
    ids32 = jnp.clip(input_ids, 0, vmax).astype(jnp.int32)
    lut32 = lookup_table.astype(jnp.int32)
    aux32 = _build_aux(multipliers, prime_mods, pad_id, max_ngram)

    s_2d = _sc_gather(lut32, ids32.reshape(-1, 128))
    out = _tc_hash(max_ngram, n_head, b, t, aux32, s_2d)
    return jnp.transpose(out, (1, 2, 0)).astype(jnp.int64)


# speedup vs baseline: 3.5821x; 1.4582x over previous
import functools

import jax
import jax.numpy as jnp
from jax import lax
from jax.experimental import pallas as pl
from jax.experimental.pallas import tpu as pltpu
from jax.experimental.pallas import tpu_sc as plsc


def _sc_gather_body(rows_per, lut_hbm, ids_hbm, out_hbm, idx_v, val_v, sem):
    nc = plsc.get_sparse_core_info().num_cores
    wid = (lax.axis_index("s") * jnp.int32(nc)
           + lax.axis_index("c")).astype(jnp.int32)
    base = wid * jnp.int32(rows_per)
    pltpu.sync_copy(ids_hbm.at[pl.ds(base, rows_per)], idx_v)
    copies = [
        pltpu.async_copy(
            lut_hbm.at[idx_v.at[jnp.int32(j)]],
            val_v.at[jnp.int32(j)], sem)
        for j in range(rows_per)
    ]
    for c in copies:
        c.wait()
    pltpu.sync_copy(val_v, out_hbm.at[pl.ds(base, rows_per)])


def _sc_gather(lut32, ids_2d):
    rows, width = ids_2d.shape
    info = plsc.get_sparse_core_info()
    rows_per = rows // (info.num_cores * info.num_subcores)
    mesh = plsc.VectorSubcoreMesh(core_axis_name="c", subcore_axis_name="s")

    return pl.kernel(
        functools.partial(_sc_gather_body, rows_per),
        out_type=jax.ShapeDtypeStruct((rows, width), jnp.int32),
        mesh=mesh,
        scratch_types=[
            pltpu.VMEM((rows_per, width), jnp.int32),
            pltpu.VMEM((rows_per, width), jnp.int32),
            pltpu.SemaphoreType.DMA,
        ],
    )(lut32, ids_2d)


def _hash_body(max_ngram, n_head, b, t, aux_ref, s_ref, out_ref):
    mask16 = jnp.uint32(0xFFFF)
    view = s_ref[...].reshape(b, t)
    pad = aux_ref[3 * max_ngram + n_head * (max_ngram - 1)]

    prods = []
    for k in range(max_ngram):
        if k == 0:
            s = view.astype(jnp.uint32)
        else:
            s = jnp.concatenate(
                [jnp.full((b, k), pad, jnp.int32), view[:, :t - k]],
                axis=1).astype(jnp.uint32)
        s0 = s & mask16
        s1_nz = (s >> 16) > 0
        mk0 = aux_ref[3 * k].astype(jnp.uint32)
        mk1 = aux_ref[3 * k + 1].astype(jnp.uint32)
        mk2 = aux_ref[3 * k + 2].astype(jnp.uint32)
        a0 = s0 * mk0
        a1 = s0 * mk1
        a2 = s0 * mk2
        b0 = jnp.where(s1_nz, mk0, jnp.uint32(0))
        b1 = jnp.where(s1_nz, mk1, jnp.uint32(0))
        b2 = jnp.where(s1_nz, mk2, jnp.uint32(0))
        c0 = a0 & mask16
        t1 = (a0 >> 16) + (a1 & mask16) + b0
        t2 = (t1 >> 16) + (a1 >> 16) + (a2 & mask16) + b1
        t3 = (t2 >> 16) + (a2 >> 16) + b2
        lo = c0 | ((t1 & mask16) << 16)
        hi = (t2 & mask16) | ((t3 & mask16) << 16)
        prods.append((hi, lo))

    mix_hi, mix_lo = prods[0]
    idx = 0
    for n in range(2, max_ngram + 1):
        mix_hi = mix_hi ^ prods[n - 1][0]
        mix_lo = mix_lo ^ prods[n - 1][1]
        hi_s = mix_hi.astype(jnp.int32)
        l1 = (mix_lo >> 16).astype(jnp.int32)
        l0 = (mix_lo & mask16).astype(jnp.int32)
        for _ in range(n_head):
            p = aux_ref[3 * max_ngram + idx]
            inv = jnp.float32(1.0) / p.astype(jnp.float32)

            def fmod31(y):
                q = (y.astype(jnp.float32) * inv).astype(jnp.int32)
                r = y - q * p
                r = jnp.where(r < 0, r + p, r)
                return jnp.where(r >= p, r - p, r)

            acc = fmod31(hi_s)
            acc = fmod31(acc << 14)
            acc = fmod31(((acc << 2) + l1) << 12)
            acc = fmod31((acc << 4) + l0)
            out_ref[idx] = acc
            idx += 1


def _tc_hash(max_ngram, n_head, b, t, aux32, s_2d, interpret=False):
    n_out = (max_ngram - 1) * n_head
    return pl.pallas_call(
        functools.partial(_hash_body, max_ngram, n_head, b, t),
        out_shape=jax.ShapeDtypeStruct((n_out, b, t), jnp.int32),
        in_specs=[
            pl.BlockSpec(memory_space=pltpu.SMEM),
            pl.BlockSpec(memory_space=pltpu.VMEM),
        ],
        out_specs=pl.BlockSpec(memory_space=pltpu.VMEM),
        interpret=interpret,
    )(aux32, s_2d)


def _build_aux(multipliers, prime_mods, pad_id, max_ngram):
    shifts = 16 * jnp.arange(3, dtype=multipliers.dtype)
    limbs = ((multipliers[:, None] >> shifts[None, :]) & 0xFFFF).reshape(-1)
    pad64 = jnp.asarray(pad_id).astype(jnp.int64)
    return jnp.concatenate([limbs, prime_mods, pad64[None]]).astype(jnp.int32)


def kernel(input_ids, lookup_table, multipliers, prime_mods, pad_id):
    b, t = input_ids.shape
    max_ngram = multipliers.shape[0]
    n_head = prime_mods.shape[0] // (max_ngram - 1)

    vmax = lookup_table.shape[0] - 1
    ids32 = jnp.clip(input_ids, 0, vmax).astype(jnp.int32)
    lut32 = lookup_table.astype(jnp.int32)
    aux32 = _build_aux(multipliers, prime_mods, pad_id, max_ngram)

    s_2d = _sc_gather(lut32, ids32.reshape(-1, 128))
    out = _tc_hash(max_ngram, n_head, b, t, aux32, s_2d)
    return jnp.transpose(out, (1, 2, 0)).astype(jnp.int64)


# speedup vs baseline: 3.5841x; 1.0006x over previous
import functools

import jax
import jax.numpy as jnp
from jax import lax
from jax.experimental import pallas as pl
from jax.experimental.pallas import tpu as pltpu
from jax.experimental.pallas import tpu_sc as plsc


def _sc_gather_body(rows_per, lut_hbm, ids_hbm, out_hbm, idx_v, val_v,
                    sem, sem_out):
    nc = plsc.get_sparse_core_info().num_cores
    wid = (lax.axis_index("s") * jnp.int32(nc)
           + lax.axis_index("c")).astype(jnp.int32)
    base = wid * jnp.int32(rows_per)
    pltpu.sync_copy(ids_hbm.at[pl.ds(base, rows_per)], idx_v)
    copies = [
        pltpu.async_copy(
            lut_hbm.at[idx_v.at[jnp.int32(j)]],
            val_v.at[jnp.int32(j)], sem)
        for j in range(rows_per)
    ]
    out_copies = []
    for j, c in enumerate(copies):
        c.wait()
        out_copies.append(
            pltpu.async_copy(val_v.at[jnp.int32(j)],
                             out_hbm.at[base + jnp.int32(j)], sem_out))
    for c in out_copies:
        c.wait()


def _sc_gather(lut32, ids_2d):
    rows, width = ids_2d.shape
    info = plsc.get_sparse_core_info()
    rows_per = rows // (info.num_cores * info.num_subcores)
    mesh = plsc.VectorSubcoreMesh(core_axis_name="c", subcore_axis_name="s")

    return pl.kernel(
        functools.partial(_sc_gather_body, rows_per),
        out_type=jax.ShapeDtypeStruct((rows, width), jnp.int32),
        mesh=mesh,
        scratch_types=[
            pltpu.VMEM((rows_per, width), jnp.int32),
            pltpu.VMEM((rows_per, width), jnp.int32),
            pltpu.SemaphoreType.DMA,
            pltpu.SemaphoreType.DMA,
        ],
    )(lut32, ids_2d)


def _hash_body(max_ngram, n_head, b, t, aux_ref, s_ref, out_ref):
    mask16 = jnp.uint32(0xFFFF)
    view = s_ref[...].reshape(b, t)
    pad = aux_ref[3 * max_ngram + n_head * (max_ngram - 1)]

    prods = []
    for k in range(max_ngram):
        if k == 0:
            s = view.astype(jnp.uint32)
        else:
            s = jnp.concatenate(
                [jnp.full((b, k), pad, jnp.int32), view[:, :t - k]],
                axis=1).astype(jnp.uint32)
        s0 = s & mask16
        s1_nz = (s >> 16) > 0
        mk0 = aux_ref[3 * k].astype(jnp.uint32)
        mk1 = aux_ref[3 * k + 1].astype(jnp.uint32)
        mk2 = aux_ref[3 * k + 2].astype(jnp.uint32)
        a0 = s0 * mk0
        a1 = s0 * mk1
        a2 = s0 * mk2
        b0 = jnp.where(s1_nz, mk0, jnp.uint32(0))
        b1 = jnp.where(s1_nz, mk1, jnp.uint32(0))
        b2 = jnp.where(s1_nz, mk2, jnp.uint32(0))
        c0 = a0 & mask16
        t1 = (a0 >> 16) + (a1 & mask16) + b0
        t2 = (t1 >> 16) + (a1 >> 16) + (a2 & mask16) + b1
        t3 = (t2 >> 16) + (a2 >> 16) + b2
        lo = c0 | ((t1 & mask16) << 16)
        hi = (t2 & mask16) | ((t3 & mask16) << 16)
        prods.append((hi, lo))

    mix_hi, mix_lo = prods[0]
    idx = 0
    for n in range(2, max_ngram + 1):
        mix_hi = mix_hi ^ prods[n - 1][0]
        mix_lo = mix_lo ^ prods[n - 1][1]
        hi_s = mix_hi.astype(jnp.int32)
        l1 = (mix_lo >> 16).astype(jnp.int32)
        l0 = (mix_lo & mask16).astype(jnp.int32)
        for _ in range(n_head):
            p = aux_ref[3 * max_ngram + idx]
            inv = jnp.float32(1.0) / p.astype(jnp.float32)

            def fmod31(y):
                q = (y.astype(jnp.float32) * inv).astype(jnp.int32)
                r = y - q * p
                r = jnp.where(r < 0, r + p, r)
                return jnp.where(r >= p, r - p, r)

            acc = fmod31(hi_s)
            acc = fmod31(acc << 14)
            acc = fmod31(((acc << 2) + l1) << 12)
            acc = fmod31((acc << 4) + l0)
            out_ref[idx] = acc
            idx += 1


def _tc_hash(max_ngram, n_head, b, t, aux32, s_2d, interpret=False):
    n_out = (max_ngram - 1) * n_head
    return pl.pallas_call(
        functools.partial(_hash_body, max_ngram, n_head, b, t),
        out_shape=jax.ShapeDtypeStruct((n_out, b, t), jnp.int32),
        in_specs=[
            pl.BlockSpec(memory_space=pltpu.SMEM),
            pl.BlockSpec(memory_space=pltpu.VMEM),
        ],
        out_specs=pl.BlockSpec(memory_space=pltpu.VMEM),
        interpret=interpret,
    )(aux32, s_2d)


def _build_aux(multipliers, prime_mods, pad_id, max_ngram):
    shifts = 16 * jnp.arange(3, dtype=multipliers.dtype)
    limbs = ((multipliers[:, None] >> shifts[None, :]) & 0xFFFF).reshape(-1)
    pad64 = jnp.asarray(pad_id).astype(jnp.int64)
    return jnp.concatenate([limbs, prime_mods, pad64[None]]).astype(jnp.int32)


def kernel(input_ids, lookup_table, multipliers, prime_mods, pad_id):
    b, t = input_ids.shape
    max_ngram = multipliers.shape[0]
    n_head = prime_mods.shape[0] // (max_ngram - 1)

    vmax = lookup_table.shape[0] - 1
    ids32 = jnp.clip(input_ids, 0, vmax).astype(jnp.int32)
    lut32 = lookup_table.astype(jnp.int32)
    aux32 = _build_aux(multipliers, prime_mods, pad_id, max_ngram)

    s_2d = _sc_gather(lut32, ids32.reshape(-1, 128))
    out = _tc_hash(max_ngram, n_head, b, t, aux32, s_2d)
    return jnp.transpose(out, (1, 2, 0)).astype(jnp.int64)
